# baseline (speedup 1.0000x reference)
import functools

import jax
import jax.numpy as jnp
from jax import lax
from jax.experimental import pallas as pl
from jax.experimental.pallas import tpu as pltpu
from jax.experimental.pallas import tpu_sc as plsc

VOCAB = 100000
DIM = 512
BATCH = 4
SEQ = 8192
NTOK = BATCH * SEQ

NC = 2
NS = 16
NW = NC * NS
TOK_PER_W = NTOK // NW
CHUNK = 80
CHUNK_LENS = [80] * 12 + [64]
CHUNK_OFFS = [80 * i for i in range(12)] + [960]
NCHUNK = len(CHUNK_OFFS)
NBUF = 3


def _embed_body(tokens_hbm, table_hbm, out_hbm, idx_v, *scratch):
    cid = lax.axis_index("c")
    sid = lax.axis_index("s")
    wid = sid * NC + cid
    base = wid * TOK_PER_W
    pltpu.sync_copy(tokens_hbm.at[pl.ds(base, TOK_PER_W)], idx_v)

    bufs = scratch[:NBUF]
    gsems = scratch[NBUF:2 * NBUF]
    osems = scratch[2 * NBUF:]
    gat = [None] * NBUF
    out = [None] * NBUF
    look = NBUF - 1
    for c in range(NCHUNK + look):
        if c < NCHUNK:
            b = c % NBUF
            if out[b] is not None:
                out[b].wait()
            n = CHUNK_LENS[c]
            gat[b] = pltpu.async_copy(
                table_hbm.at[idx_v.at[pl.ds(CHUNK_OFFS[c], n)]],
                bufs[b].at[pl.ds(0, n)], gsems[b])
        d = c - look
        if d >= 0:
            b = d % NBUF
            gat[b].wait()
            n = CHUNK_LENS[d]
            out[b] = pltpu.async_copy(
                bufs[b].at[pl.ds(0, n)],
                out_hbm.at[pl.ds(base + CHUNK_OFFS[d], n)],
                osems[b])
    for b in range(NBUF):
        if out[b] is not None:
            out[b].wait()


@jax.jit
def _embed(tokens_flat, table):
    mesh = plsc.VectorSubcoreMesh(core_axis_name="c", subcore_axis_name="s")
    return pl.kernel(
        _embed_body,
        out_type=jax.ShapeDtypeStruct((NTOK, DIM), jnp.float32),
        mesh=mesh,
        scratch_types=(
            [pltpu.VMEM((TOK_PER_W,), jnp.int32)]
            + [pltpu.VMEM((CHUNK, DIM), jnp.float32)] * NBUF
            + [pltpu.SemaphoreType.DMA] * (2 * NBUF)
        ),
    )(tokens_flat, table)


def kernel(tokens, tok_embeddings_weight):
    tokens_flat = tokens.reshape(-1).astype(jnp.int32)
    out = _embed(tokens_flat, tok_embeddings_weight)
    return out.reshape(BATCH, SEQ, DIM)


# speedup vs baseline: 1.0013x; 1.0013x over previous
import jax
import jax.numpy as jnp
from jax import lax
from jax.experimental import pallas as pl
from jax.experimental.pallas import tpu as pltpu
from jax.experimental.pallas import tpu_sc as plsc

VOCAB = 100000
DIM = 512
BATCH = 4
SEQ = 8192
NTOK = BATCH * SEQ

NC = 2
NS = 16
NW = NC * NS
TOK_PER_W = NTOK // NW
CHUNK = 80
CHUNK_LENS = [80] * 12 + [64]
CHUNK_OFFS = [80 * i for i in range(12)] + [960]
NCHUNK = len(CHUNK_OFFS)
NBUF = 3


def _embed_body(tokens_hbm, table_hbm, out_hbm, idx_v, *scratch):
    cid = lax.axis_index("c")
    sid = lax.axis_index("s")
    wid = sid * NC + cid
    base = wid * TOK_PER_W
    pltpu.sync_copy(tokens_hbm.at[pl.ds(base, TOK_PER_W)], idx_v)

    bufs = scratch[:NBUF]
    gsems = scratch[NBUF:2 * NBUF]
    osems = scratch[2 * NBUF:]
    gat = [None] * NBUF
    out = [None] * NBUF
    look = NBUF - 1
    for c in range(NCHUNK + look):
        if c < NCHUNK:
            b = c % NBUF
            if out[b] is not None:
                out[b].wait()
            n = CHUNK_LENS[c]
            gat[b] = pltpu.async_copy(
                table_hbm.at[idx_v.at[pl.ds(CHUNK_OFFS[c], n)]],
                bufs[b].at[pl.ds(0, n)], gsems[b])
        d = c - look
        if d >= 0:
            b = d % NBUF
            gat[b].wait()
            n = CHUNK_LENS[d]
            out[b] = pltpu.async_copy(
                bufs[b].at[pl.ds(0, n)],
                out_hbm.at[pl.ds(base + CHUNK_OFFS[d], n)],
                osems[b])
    for b in range(NBUF):
        if out[b] is not None:
            out[b].wait()


@jax.jit
def _embed(tokens_flat, table):
    mesh = plsc.VectorSubcoreMesh(core_axis_name="c", subcore_axis_name="s")
    return pl.kernel(
        _embed_body,
        out_type=jax.ShapeDtypeStruct((NTOK, DIM), jnp.float32),
        mesh=mesh,
        scratch_types=(
            [pltpu.VMEM((TOK_PER_W,), jnp.int32)]
            + [pltpu.VMEM((CHUNK, DIM), jnp.float32)] * NBUF
            + [pltpu.SemaphoreType.DMA] * (2 * NBUF)
        ),
    )(tokens_flat, table)


def kernel(tokens, tok_embeddings_weight):
    tokens_flat = tokens.reshape(-1).astype(jnp.int32)
    out = _embed(tokens_flat, tok_embeddings_weight)
    return out.reshape(BATCH, SEQ, DIM)
